# baseline (speedup 1.0000x reference)
import functools

import jax
import jax.numpy as jnp
from jax import lax
from jax.experimental import pallas as pl
from jax.experimental.pallas import tpu as pltpu
from jax.experimental.pallas import tpu_sc as plsc

N = 16384
D = 512
H = 80
F = 256
K = 200

BLK = 1024
NBLK = N // BLK
INF = float("inf")

SC_NC = 2
SC_NS = 16
SC_NW = SC_NC * SC_NS
KPAD = 256
ROWS_W = KPAD // SC_NW


def _h_body(x_ref, w_ref, b_ref, o_ref):
    p0 = jnp.dot(x_ref[:, 0:256], w_ref[0:256, :],
                 preferred_element_type=jnp.float32)
    p1 = jnp.dot(x_ref[:, 256:512], w_ref[256:512, :],
                 preferred_element_type=jnp.float32)
    o_ref[...] = (b_ref[...] + p0) + p1


def _fu_body(h_ref, w_ref, b_ref, o_ref):
    o_ref[...] = jnp.dot(h_ref[...], w_ref[...],
                         preferred_element_type=jnp.float32) + b_ref[...]


def _tree8(acc):
    t = acc[4:8, :] + acc[0:4, :]
    u = t[2:4, :] + t[0:2, :]
    return u[1:2, :] + u[0:1, :]


def _norm2_feats(sq_slab):
    t3 = sq_slab.T.reshape(32, 8, 128)
    acc = t3[0]
    for v in range(1, 32):
        acc = acc + t3[v]
    return _tree8(acc)


def _norm2_scan(sq_slab):
    part = []
    for b in range(2):
        t3 = sq_slab[:, b * 128:(b + 1) * 128].T.reshape(16, 8, 128)
        acc = t3[0]
        for v in range(1, 16):
            acc = acc + t3[v]
        part.append(_tree8(acc))
    return part[0] + part[1]


def _feats_body(fu_ref, o_ref):
    fub = fu_ref[...]
    for sl in range(BLK // 128):
        fus = fub[sl * 128:(sl + 1) * 128, :]
        nr = jnp.sqrt(_norm2_feats(fus * fus))
        o_ref[sl * 128:(sl + 1) * 128, :] = fus / nr.T


def _mu_body(c_ref, o_ref):
    c2 = c_ref[...].reshape(2, 128)
    sq = c2 * c2
    part = sq[1:2, :] + sq[0:1, :]
    o_ref[...] = c_ref[...] / jnp.sqrt(jnp.sum(part))


def _herd_body(f3_ref, mu_ref, idx_ref, mask3):
    mu = mu_ref[...]
    mask3[...] = jnp.zeros((N // 128, 1, 128), jnp.float32)
    lane = lax.broadcasted_iota(jnp.int32, (1, 128), 1)
    row = lax.broadcasted_iota(jnp.int32, (128, 1), 0)
    lin8 = (lax.broadcasted_iota(jnp.int32, (8, 128), 0) * 128
            + lax.broadcasted_iota(jnp.int32, (8, 128), 1))

    def step(k, carry):
        s, rec = carry
        c = 1.0 / (k.astype(jnp.float32) + 1.0)

        def slab(sl, best_carry):
            best, bidx = best_carry
            f = f3_ref[sl]
            d = mu - c * (f + s)
            dist = jnp.sqrt(_norm2_scan(d * d))
            dist = jnp.where(mask3[sl] > 0.0, INF, dist)
            m = jnp.min(dist)
            li = jnp.min(jnp.where(dist == m, lane, N))
            upd = m < best
            best = jnp.where(upd, m, best)
            bidx = jnp.where(upd, sl * 128 + li, bidx)
            return best, bidx

        _, sel = lax.fori_loop(0, N // 128, slab, (jnp.float32(INF),
                                                   jnp.int32(0)))
        rb = sel // 128
        ro = sel - rb * 128
        mrow = mask3[rb]
        mask3[rb] = jnp.where(lane == ro, 1.0, mrow)
        fb = f3_ref[rb]
        fsel = jnp.sum(jnp.where(row == ro, fb, 0.0), axis=0, keepdims=True)
        s = s + fsel
        rec = jnp.where(lin8 == k, sel, rec)
        return s, rec

    _, rec = lax.fori_loop(0, K, step, (jnp.zeros((1, F), jnp.float32),
                                        jnp.zeros((8, 128), jnp.int32)))
    idx_ref[...] = rec


@functools.cache
def _sc_gather_kernel():
    @functools.partial(
        pl.kernel,
        out_type=jax.ShapeDtypeStruct((KPAD, D), jnp.float32),
        mesh=plsc.VectorSubcoreMesh(core_axis_name="c", subcore_axis_name="s"),
        scratch_types=[
            pltpu.VMEM((ROWS_W,), jnp.int32),
            pltpu.VMEM((ROWS_W, D), jnp.float32),
            pltpu.SemaphoreType.DMA,
        ],
    )
    def _sc_gather(x_hbm, idx_hbm, out_hbm, idx_v, rows_v, sem):
        wid = lax.axis_index("s") * SC_NC + lax.axis_index("c")
        base = wid * ROWS_W
        pltpu.sync_copy(idx_hbm.at[pl.ds(base, ROWS_W)], idx_v)
        pltpu.async_copy(x_hbm.at[idx_v], rows_v, sem).wait()
        pltpu.sync_copy(rows_v, out_hbm.at[pl.ds(base, ROWS_W)])

    return _sc_gather


def kernel(X, y, W1, b1, W2, b2):
    h = pl.pallas_call(
        _h_body, grid=(NBLK,),
        in_specs=[pl.BlockSpec((BLK, D), lambda i: (i, 0)),
                  pl.BlockSpec((D, H), lambda i: (0, 0)),
                  pl.BlockSpec((1, H), lambda i: (0, 0))],
        out_specs=pl.BlockSpec((BLK, H), lambda i: (i, 0)),
        out_shape=jax.ShapeDtypeStruct((N, H), jnp.float32),
        compiler_params=pltpu.CompilerParams(
            dimension_semantics=("arbitrary",)),
    )(X, W1, b1.reshape(1, H))

    fu = pl.pallas_call(
        _fu_body, grid=(NBLK,),
        in_specs=[pl.BlockSpec((BLK, H), lambda i: (i, 0)),
                  pl.BlockSpec((H, F), lambda i: (0, 0)),
                  pl.BlockSpec((1, F), lambda i: (0, 0))],
        out_specs=pl.BlockSpec((BLK, F), lambda i: (i, 0)),
        out_shape=jax.ShapeDtypeStruct((N, F), jnp.float32),
        compiler_params=pltpu.CompilerParams(
            dimension_semantics=("arbitrary",)),
    )(h, W2, b2.reshape(1, F))

    feats = pl.pallas_call(
        _feats_body, grid=(NBLK,),
        in_specs=[pl.BlockSpec((BLK, F), lambda i: (i, 0))],
        out_specs=pl.BlockSpec((BLK, F), lambda i: (i, 0)),
        out_shape=jax.ShapeDtypeStruct((N, F), jnp.float32),
        compiler_params=pltpu.CompilerParams(
            dimension_semantics=("arbitrary",)),
    )(fu)

    cm = jnp.mean(feats, axis=0, keepdims=True)

    mu = pl.pallas_call(
        _mu_body,
        in_specs=[pl.BlockSpec((1, F), lambda: (0, 0))],
        out_specs=pl.BlockSpec((1, F), lambda: (0, 0)),
        out_shape=jax.ShapeDtypeStruct((1, F), jnp.float32),
    )(cm)

    f3 = feats.reshape(N // 128, 128, F)
    idx8 = pl.pallas_call(
        _herd_body,
        in_specs=[pl.BlockSpec((N // 128, 128, F), lambda: (0, 0, 0)),
                  pl.BlockSpec((1, F), lambda: (0, 0))],
        out_specs=pl.BlockSpec((8, 128), lambda: (0, 0)),
        out_shape=jax.ShapeDtypeStruct((8, 128), jnp.int32),
        scratch_shapes=[pltpu.VMEM((N // 128, 1, 128), jnp.float32)],
    )(f3, mu)

    indices = idx8.reshape(-1)[:K]
    idx_pad = jnp.concatenate([indices, jnp.zeros((KPAD - K,), jnp.int32)])
    exemplar_x = _sc_gather_kernel()(X, idx_pad)
    return indices, exemplar_x[:K]


# speedup vs baseline: 1.9251x; 1.9251x over previous
import functools

import jax
import jax.numpy as jnp
from jax import lax
from jax.experimental import pallas as pl
from jax.experimental.pallas import tpu as pltpu
from jax.experimental.pallas import tpu_sc as plsc

N = 16384
D = 512
H = 80
F = 256
K = 200

BLK = 1024
NBLK = N // BLK
INF = float("inf")

SC_NC = 2
SC_NS = 16
SC_NW = SC_NC * SC_NS
KPAD = 256
ROWS_W = KPAD // SC_NW


def _h_body(x_ref, w_ref, b_ref, o_ref):
    p0 = jnp.dot(x_ref[:, 0:256], w_ref[0:256, :],
                 preferred_element_type=jnp.float32)
    p1 = jnp.dot(x_ref[:, 256:512], w_ref[256:512, :],
                 preferred_element_type=jnp.float32)
    o_ref[...] = (b_ref[...] + p0) + p1


def _fu_body(h_ref, w_ref, b_ref, o_ref):
    o_ref[...] = jnp.dot(h_ref[...], w_ref[...],
                         preferred_element_type=jnp.float32) + b_ref[...]


def _tree8(acc):
    t = acc[4:8, :] + acc[0:4, :]
    u = t[2:4, :] + t[0:2, :]
    return u[1:2, :] + u[0:1, :]


def _norm2_feats(sq_slab):
    t3 = sq_slab.T.reshape(32, 8, 128)
    acc = t3[0]
    for v in range(1, 32):
        acc = acc + t3[v]
    return _tree8(acc)


def _norm2_scan(sq_slab):
    part = []
    for b in range(2):
        t3 = sq_slab[:, b * 128:(b + 1) * 128].T.reshape(16, 8, 128)
        acc = t3[0]
        for v in range(1, 16):
            acc = acc + t3[v]
        part.append(_tree8(acc))
    return part[0] + part[1]


def _feats_body(fu_ref, o_ref):
    fub = fu_ref[...]
    for sl in range(BLK // 128):
        fus = fub[sl * 128:(sl + 1) * 128, :]
        nr = jnp.sqrt(_norm2_feats(fus * fus))
        o_ref[sl * 128:(sl + 1) * 128, :] = fus / nr.T


def _mu_body(c_ref, o_ref):
    c2 = c_ref[...].reshape(2, 128)
    sq = c2 * c2
    part = sq[1:2, :] + sq[0:1, :]
    o_ref[...] = c_ref[...] / jnp.sqrt(jnp.sum(part))


def _herd_body(f3_ref, mu_ref, idx_ref, mask3):
    mu = mu_ref[...]
    mask3[...] = jnp.zeros((N // 128, 1, 128), jnp.float32)
    lane = lax.broadcasted_iota(jnp.int32, (1, 128), 1)
    row = lax.broadcasted_iota(jnp.int32, (128, 1), 0)
    lin8 = (lax.broadcasted_iota(jnp.int32, (8, 128), 0) * 128
            + lax.broadcasted_iota(jnp.int32, (8, 128), 1))

    def step(k, carry):
        s, rec = carry
        c = 1.0 / (k.astype(jnp.float32) + 1.0)

        def slab(sl, best_carry):
            best, bidx = best_carry
            f = f3_ref[sl]
            d = mu - c * (f + s)
            dist = jnp.sqrt(_norm2_scan(d * d))
            dist = jnp.where(mask3[sl] > 0.0, INF, dist)
            m = jnp.min(dist)
            li = jnp.min(jnp.where(dist == m, lane, N))
            upd = m < best
            best = jnp.where(upd, m, best)
            bidx = jnp.where(upd, sl * 128 + li, bidx)
            return best, bidx

        _, sel = lax.fori_loop(0, N // 128, slab, (jnp.float32(INF),
                                                   jnp.int32(0)), unroll=8)
        rb = sel // 128
        ro = sel - rb * 128
        mrow = mask3[rb]
        mask3[rb] = jnp.where(lane == ro, 1.0, mrow)
        fb = f3_ref[rb]
        fsel = jnp.sum(jnp.where(row == ro, fb, 0.0), axis=0, keepdims=True)
        s = s + fsel
        rec = jnp.where(lin8 == k, sel, rec)
        return s, rec

    _, rec = lax.fori_loop(0, K, step, (jnp.zeros((1, F), jnp.float32),
                                        jnp.zeros((8, 128), jnp.int32)))
    idx_ref[...] = rec


@functools.cache
def _sc_gather_kernel():
    @functools.partial(
        pl.kernel,
        out_type=jax.ShapeDtypeStruct((KPAD, D), jnp.float32),
        mesh=plsc.VectorSubcoreMesh(core_axis_name="c", subcore_axis_name="s"),
        scratch_types=[
            pltpu.VMEM((ROWS_W,), jnp.int32),
            pltpu.VMEM((ROWS_W, D), jnp.float32),
            pltpu.SemaphoreType.DMA,
        ],
    )
    def _sc_gather(x_hbm, idx_hbm, out_hbm, idx_v, rows_v, sem):
        wid = lax.axis_index("s") * SC_NC + lax.axis_index("c")
        base = wid * ROWS_W
        pltpu.sync_copy(idx_hbm.at[pl.ds(base, ROWS_W)], idx_v)
        pltpu.async_copy(x_hbm.at[idx_v], rows_v, sem).wait()
        pltpu.sync_copy(rows_v, out_hbm.at[pl.ds(base, ROWS_W)])

    return _sc_gather


def kernel(X, y, W1, b1, W2, b2):
    h = pl.pallas_call(
        _h_body, grid=(NBLK,),
        in_specs=[pl.BlockSpec((BLK, D), lambda i: (i, 0)),
                  pl.BlockSpec((D, H), lambda i: (0, 0)),
                  pl.BlockSpec((1, H), lambda i: (0, 0))],
        out_specs=pl.BlockSpec((BLK, H), lambda i: (i, 0)),
        out_shape=jax.ShapeDtypeStruct((N, H), jnp.float32),
        compiler_params=pltpu.CompilerParams(
            dimension_semantics=("arbitrary",)),
    )(X, W1, b1.reshape(1, H))

    fu = pl.pallas_call(
        _fu_body, grid=(NBLK,),
        in_specs=[pl.BlockSpec((BLK, H), lambda i: (i, 0)),
                  pl.BlockSpec((H, F), lambda i: (0, 0)),
                  pl.BlockSpec((1, F), lambda i: (0, 0))],
        out_specs=pl.BlockSpec((BLK, F), lambda i: (i, 0)),
        out_shape=jax.ShapeDtypeStruct((N, F), jnp.float32),
        compiler_params=pltpu.CompilerParams(
            dimension_semantics=("arbitrary",)),
    )(h, W2, b2.reshape(1, F))

    feats = pl.pallas_call(
        _feats_body, grid=(NBLK,),
        in_specs=[pl.BlockSpec((BLK, F), lambda i: (i, 0))],
        out_specs=pl.BlockSpec((BLK, F), lambda i: (i, 0)),
        out_shape=jax.ShapeDtypeStruct((N, F), jnp.float32),
        compiler_params=pltpu.CompilerParams(
            dimension_semantics=("arbitrary",)),
    )(fu)

    cm = jnp.mean(feats, axis=0, keepdims=True)

    mu = pl.pallas_call(
        _mu_body,
        in_specs=[pl.BlockSpec((1, F), lambda: (0, 0))],
        out_specs=pl.BlockSpec((1, F), lambda: (0, 0)),
        out_shape=jax.ShapeDtypeStruct((1, F), jnp.float32),
    )(cm)

    f3 = feats.reshape(N // 128, 128, F)
    idx8 = pl.pallas_call(
        _herd_body,
        in_specs=[pl.BlockSpec((N // 128, 128, F), lambda: (0, 0, 0)),
                  pl.BlockSpec((1, F), lambda: (0, 0))],
        out_specs=pl.BlockSpec((8, 128), lambda: (0, 0)),
        out_shape=jax.ShapeDtypeStruct((8, 128), jnp.int32),
        scratch_shapes=[pltpu.VMEM((N // 128, 1, 128), jnp.float32)],
    )(f3, mu)

    indices = idx8.reshape(-1)[:K]
    idx_pad = jnp.concatenate([indices, jnp.zeros((KPAD - K,), jnp.int32)])
    exemplar_x = _sc_gather_kernel()(X, idx_pad)
    return indices, exemplar_x[:K]


# speedup vs baseline: 2.0565x; 1.0683x over previous
import functools

import jax
import jax.numpy as jnp
from jax import lax
from jax.experimental import pallas as pl
from jax.experimental.pallas import tpu as pltpu
from jax.experimental.pallas import tpu_sc as plsc

N = 16384
D = 512
H = 80
F = 256
K = 200

BLK = 1024
NBLK = N // BLK
INF = float("inf")

SC_NC = 2
SC_NS = 16
SC_NW = SC_NC * SC_NS
KPAD = 256
ROWS_W = KPAD // SC_NW


def _h_body(x_ref, w_ref, b_ref, o_ref):
    p0 = jnp.dot(x_ref[:, 0:256], w_ref[0:256, :],
                 preferred_element_type=jnp.float32)
    p1 = jnp.dot(x_ref[:, 256:512], w_ref[256:512, :],
                 preferred_element_type=jnp.float32)
    o_ref[...] = (b_ref[...] + p0) + p1


def _fu_body(h_ref, w_ref, b_ref, o_ref):
    o_ref[...] = jnp.dot(h_ref[...], w_ref[...],
                         preferred_element_type=jnp.float32) + b_ref[...]


def _tree8(acc):
    t = acc[4:8, :] + acc[0:4, :]
    u = t[2:4, :] + t[0:2, :]
    return u[1:2, :] + u[0:1, :]


def _norm2_feats(sq_slab):
    t3 = sq_slab.T.reshape(32, 8, 128)
    acc = t3[0]
    for v in range(1, 32):
        acc = acc + t3[v]
    return _tree8(acc)


def _norm2_scan(sq_slab):
    part = []
    for b in range(2):
        t3 = sq_slab[:, b * 128:(b + 1) * 128].T.reshape(16, 8, 128)
        acc = t3[0]
        for v in range(1, 16):
            acc = acc + t3[v]
        part.append(_tree8(acc))
    return part[0] + part[1]


def _feats_body(fu_ref, o_ref):
    fub = fu_ref[...]
    for sl in range(BLK // 128):
        fus = fub[sl * 128:(sl + 1) * 128, :]
        nr = jnp.sqrt(_norm2_feats(fus * fus))
        o_ref[sl * 128:(sl + 1) * 128, :] = fus / nr.T


def _mu_body(c_ref, o_ref):
    c2 = c_ref[...].reshape(2, 128)
    sq = c2 * c2
    part = sq[1:2, :] + sq[0:1, :]
    o_ref[...] = c_ref[...] / jnp.sqrt(jnp.sum(part))


def _herd_body(f3_ref, mu_ref, idx_ref, mask3):
    mu = mu_ref[...]
    mask3[...] = jnp.zeros((N // 128, 1, 128), jnp.float32)
    lane = lax.broadcasted_iota(jnp.int32, (1, 128), 1)
    row = lax.broadcasted_iota(jnp.int32, (128, 1), 0)
    lin8 = (lax.broadcasted_iota(jnp.int32, (8, 128), 0) * 128
            + lax.broadcasted_iota(jnp.int32, (8, 128), 1))

    def step(k, carry):
        s, rec = carry
        c = 1.0 / (k.astype(jnp.float32) + 1.0)

        def slab(sl, best_carry):
            best, bidx = best_carry
            f = f3_ref[sl]
            d = mu - c * (f + s)
            dist = jnp.sqrt(_norm2_scan(d * d))
            dist = jnp.where(mask3[sl] > 0.0, INF, dist)
            m = jnp.min(dist)
            li = jnp.min(jnp.where(dist == m, lane, N))
            upd = m < best
            best = jnp.where(upd, m, best)
            bidx = jnp.where(upd, sl * 128 + li, bidx)
            return best, bidx

        _, sel = lax.fori_loop(0, N // 128, slab, (jnp.float32(INF),
                                                   jnp.int32(0)), unroll=16)
        rb = sel // 128
        ro = sel - rb * 128
        mrow = mask3[rb]
        mask3[rb] = jnp.where(lane == ro, 1.0, mrow)
        fb = f3_ref[rb]
        fsel = jnp.sum(jnp.where(row == ro, fb, 0.0), axis=0, keepdims=True)
        s = s + fsel
        rec = jnp.where(lin8 == k, sel, rec)
        return s, rec

    _, rec = lax.fori_loop(0, K, step, (jnp.zeros((1, F), jnp.float32),
                                        jnp.zeros((8, 128), jnp.int32)))
    idx_ref[...] = rec


@functools.cache
def _sc_gather_kernel():
    @functools.partial(
        pl.kernel,
        out_type=jax.ShapeDtypeStruct((KPAD, D), jnp.float32),
        mesh=plsc.VectorSubcoreMesh(core_axis_name="c", subcore_axis_name="s"),
        scratch_types=[
            pltpu.VMEM((ROWS_W,), jnp.int32),
            pltpu.VMEM((ROWS_W, D), jnp.float32),
            pltpu.SemaphoreType.DMA,
        ],
    )
    def _sc_gather(x_hbm, idx_hbm, out_hbm, idx_v, rows_v, sem):
        wid = lax.axis_index("s") * SC_NC + lax.axis_index("c")
        base = wid * ROWS_W
        pltpu.sync_copy(idx_hbm.at[pl.ds(base, ROWS_W)], idx_v)
        pltpu.async_copy(x_hbm.at[idx_v], rows_v, sem).wait()
        pltpu.sync_copy(rows_v, out_hbm.at[pl.ds(base, ROWS_W)])

    return _sc_gather


def kernel(X, y, W1, b1, W2, b2):
    h = pl.pallas_call(
        _h_body, grid=(NBLK,),
        in_specs=[pl.BlockSpec((BLK, D), lambda i: (i, 0)),
                  pl.BlockSpec((D, H), lambda i: (0, 0)),
                  pl.BlockSpec((1, H), lambda i: (0, 0))],
        out_specs=pl.BlockSpec((BLK, H), lambda i: (i, 0)),
        out_shape=jax.ShapeDtypeStruct((N, H), jnp.float32),
        compiler_params=pltpu.CompilerParams(
            dimension_semantics=("arbitrary",)),
    )(X, W1, b1.reshape(1, H))

    fu = pl.pallas_call(
        _fu_body, grid=(NBLK,),
        in_specs=[pl.BlockSpec((BLK, H), lambda i: (i, 0)),
                  pl.BlockSpec((H, F), lambda i: (0, 0)),
                  pl.BlockSpec((1, F), lambda i: (0, 0))],
        out_specs=pl.BlockSpec((BLK, F), lambda i: (i, 0)),
        out_shape=jax.ShapeDtypeStruct((N, F), jnp.float32),
        compiler_params=pltpu.CompilerParams(
            dimension_semantics=("arbitrary",)),
    )(h, W2, b2.reshape(1, F))

    feats = pl.pallas_call(
        _feats_body, grid=(NBLK,),
        in_specs=[pl.BlockSpec((BLK, F), lambda i: (i, 0))],
        out_specs=pl.BlockSpec((BLK, F), lambda i: (i, 0)),
        out_shape=jax.ShapeDtypeStruct((N, F), jnp.float32),
        compiler_params=pltpu.CompilerParams(
            dimension_semantics=("arbitrary",)),
    )(fu)

    cm = jnp.mean(feats, axis=0, keepdims=True)

    mu = pl.pallas_call(
        _mu_body,
        in_specs=[pl.BlockSpec((1, F), lambda: (0, 0))],
        out_specs=pl.BlockSpec((1, F), lambda: (0, 0)),
        out_shape=jax.ShapeDtypeStruct((1, F), jnp.float32),
    )(cm)

    f3 = feats.reshape(N // 128, 128, F)
    idx8 = pl.pallas_call(
        _herd_body,
        in_specs=[pl.BlockSpec((N // 128, 128, F), lambda: (0, 0, 0)),
                  pl.BlockSpec((1, F), lambda: (0, 0))],
        out_specs=pl.BlockSpec((8, 128), lambda: (0, 0)),
        out_shape=jax.ShapeDtypeStruct((8, 128), jnp.int32),
        scratch_shapes=[pltpu.VMEM((N // 128, 1, 128), jnp.float32)],
    )(f3, mu)

    indices = idx8.reshape(-1)[:K]
    idx_pad = jnp.concatenate([indices, jnp.zeros((KPAD - K,), jnp.int32)])
    exemplar_x = _sc_gather_kernel()(X, idx_pad)
    return indices, exemplar_x[:K]
